# baseline (speedup 1.0000x reference)
import functools

import jax
import jax.numpy as jnp
from jax import lax
from jax.experimental import pallas as pl
from jax.experimental.pallas import tpu as pltpu
from jax.experimental.pallas import tpu_sc as plsc

NC, NS, L = 2, 16, 16
NW = NC * NS
B, S, V, D = 4, 2048, 100000, 768
S_SC = 1280
S_TC = S - S_SC
N_SC = B * S_SC
N_TC = B * S_TC
TPW = N_SC // NW
C = 32
NCH = TPW // C
PR = C // B
NV = D // L
HR = 4
PEW = PR * D
AUX = PEW + C * 2 * L
R_TC = 32
G_TC = N_TC // R_TC


def _sc_body(idx_hbm, oidx_hbm, aux_hbm, segd_hbm, tok_hbm,
             out_hbm, idx_v, oidx_v, aux_v, segd_v, tok_v, res_v,
             tok_sem, aux_sem, out_sem, misc_sem):
    wid = lax.axis_index("s") * NC + lax.axis_index("c")

    cd_idx = pltpu.async_copy(idx_hbm.at[wid], idx_v, misc_sem)
    cd_oidx = pltpu.async_copy(oidx_hbm.at[wid], oidx_v, misc_sem)
    cd_segd = pltpu.async_copy(segd_hbm, segd_v, misc_sem)
    cd_idx.wait()

    def start_in(g):
        slot = lax.rem(g, 2)
        pltpu.async_copy(
            tok_hbm.at[idx_v.at[g]], tok_v.at[slot], tok_sem.at[slot])
        pltpu.async_copy(
            aux_hbm.at[wid, g], aux_v.at[slot], aux_sem.at[slot])

    def wait_in(g, slot):
        pltpu.make_async_copy(
            tok_hbm.at[idx_v.at[g]], tok_v.at[slot], tok_sem.at[slot]).wait()
        pltpu.make_async_copy(
            aux_hbm.at[wid, g], aux_v.at[slot], aux_sem.at[slot]).wait()

    def start_out(g, slot):
        pltpu.async_copy(
            res_v.at[slot], out_hbm.at[oidx_v.at[g]], out_sem.at[slot])

    def wait_out(g, slot):
        pltpu.make_async_copy(
            res_v.at[slot], out_hbm.at[oidx_v.at[g]], out_sem.at[slot]).wait()

    def compute(slot):
        for h in range(C // HR):
            r0 = h * HR
            was = tuple(
                aux_v[slot, pl.ds(PEW + (r0 + i) * 2 * L, L)]
                for i in range(HR))
            wbs = tuple(
                aux_v[slot, pl.ds(PEW + (r0 + i) * 2 * L + L, L)]
                for i in range(HR))

            def jbody(j, carry):
                was_, wbs_ = carry
                off = j * L
                a1 = segd_v[pl.ds(off, L)]
                a2 = segd_v[pl.ds(D + off, L)]
                for i in range(HR):
                    row = r0 + i
                    t = tok_v[slot, row, pl.ds(off, L)]
                    p = aux_v[slot, pl.ds((row // B) * D + off, L)]
                    res_v[slot, row, pl.ds(off, L)] = (
                        t + p + was_[i] * a1 + wbs_[i] * a2)
                return was_, wbs_

            plsc.parallel_loop(0, NV, 1, unroll=2, carry=(was, wbs))(jbody)

    start_in(0)
    start_in(1)
    cd_oidx.wait()
    cd_segd.wait()

    def gbody(g, carry):
        slot = lax.rem(g, 2)
        wait_in(g, slot)

        @pl.when(g >= 2)
        def _():
            wait_out(g - 2, slot)

        compute(slot)
        start_out(g, slot)

        @pl.when(g + 2 < NCH)
        def _():
            start_in(g + 2)

        return carry

    lax.fori_loop(0, NCH, gbody, 0)
    wait_out(NCH - 2, 0)
    wait_out(NCH - 1, 1)


_sc_call = functools.partial(
    pl.kernel,
    out_type=jax.ShapeDtypeStruct((B * S, D), jnp.float32),
    mesh=plsc.VectorSubcoreMesh(core_axis_name="c", subcore_axis_name="s"),
    scratch_types=[
        pltpu.VMEM((NCH, C), jnp.int32),
        pltpu.VMEM((NCH, C), jnp.int32),
        pltpu.VMEM((2, AUX), jnp.float32),
        pltpu.VMEM((2 * D,), jnp.float32),
        pltpu.VMEM((2, C, D), jnp.float32),
        pltpu.VMEM((2, C, D), jnp.float32),
        pltpu.SemaphoreType.DMA((2,)),
        pltpu.SemaphoreType.DMA((2,)),
        pltpu.SemaphoreType.DMA((2,)),
        pltpu.SemaphoreType.DMA,
    ],
)(_sc_body)


def _tc_body(idx_ref, pe_ref, w2_ref, segd_ref, tok_ref, out_ref,
             rowbuf, sem):
    g = pl.program_id(0)
    slot = lax.rem(g, 2)
    nxt = lax.rem(g + 1, 2)

    def issue(k, kslot):
        base = k * R_TC
        for r in range(R_TC):
            pltpu.make_async_copy(
                tok_ref.at[idx_ref[base + r]], rowbuf.at[kslot, r],
                sem.at[kslot]).start()

    @pl.when(g == 0)
    def _():
        issue(0, 0)

    @pl.when(g + 1 < G_TC)
    def _():
        issue(g + 1, nxt)

    pltpu.make_async_copy(
        tok_ref.at[pl.ds(0, R_TC)], rowbuf.at[slot], sem.at[slot]).wait()

    wa = w2_ref[:, 0:1]
    wb = w2_ref[:, 1:2]
    a1 = segd_ref[0:1, :]
    a2 = segd_ref[1:2, :]
    out_ref[...] = rowbuf[slot] + pe_ref[...] + wa * a1 + wb * a2


_tc_grid_spec = pltpu.PrefetchScalarGridSpec(
    num_scalar_prefetch=1,
    grid=(G_TC,),
    in_specs=[
        pl.BlockSpec((R_TC, D), lambda g, idx: (lax.rem(g, S_TC // R_TC), 0)),
        pl.BlockSpec((R_TC, 128), lambda g, idx: (g, 0)),
        pl.BlockSpec((2, D), lambda g, idx: (0, 0)),
        pl.BlockSpec(memory_space=pl.ANY),
    ],
    out_specs=pl.BlockSpec((R_TC, D), lambda g, idx: (g, 0)),
    scratch_shapes=[
        pltpu.VMEM((2, R_TC, D), jnp.float32),
        pltpu.SemaphoreType.DMA((2,)),
    ],
)

_tc_call = pl.pallas_call(
    _tc_body,
    grid_spec=_tc_grid_spec,
    out_shape=jax.ShapeDtypeStruct((N_TC, D), jnp.float32),
)


def kernel(sequence, segment_labels, tok_table, seg_table, pe):
    sequence = sequence.astype(jnp.int32)
    segment_labels = segment_labels.astype(jnp.int32)
    pef = pe.reshape(S, D) + seg_table[0]
    d1 = seg_table[1] - seg_table[0]
    d2 = seg_table[2] - seg_table[1]

    seq_sm = sequence[:, :S_SC].T.reshape(NW, NCH, C)
    lbl_sm = segment_labels[:, :S_SC].T.reshape(NW, TPW)
    w = jnp.broadcast_to(
        jnp.stack([(lbl_sm >= 1), (lbl_sm >= 2)], axis=-1)
        .astype(jnp.float32)[..., None],
        (NW, TPW, 2, L)).reshape(NW, NCH, C * 2 * L)
    tp = jnp.arange(N_SC, dtype=jnp.int32)
    oidx = ((tp % B) * S + tp // B).reshape(NW, NCH, C)
    segd = jnp.concatenate([d1, d2])
    pe5 = pef[:S_SC].reshape(NW, NCH, PEW)
    aux = jnp.concatenate([pe5, w], axis=-1)
    out_sc = _sc_call(seq_sm, oidx, aux, segd, tok_table)

    idx_tc = sequence[:, S_SC:].reshape(-1)
    lbl_tc = segment_labels[:, S_SC:].reshape(-1)
    w2 = jnp.pad(
        jnp.stack([(lbl_tc >= 1), (lbl_tc >= 2)], axis=-1)
        .astype(jnp.float32),
        ((0, 0), (0, 126)))
    segd2 = jnp.stack([d1, d2])
    out_tc = _tc_call(idx_tc, pef[S_SC:], w2, segd2, tok_table)

    out = out_sc.reshape(B, S, D)
    return out.at[:, S_SC:, :].set(out_tc.reshape(B, S_TC, D))


# speedup vs baseline: 1.8432x; 1.8432x over previous
import functools

import jax
import jax.numpy as jnp
from jax import lax
from jax.experimental import pallas as pl
from jax.experimental.pallas import tpu as pltpu
from jax.experimental.pallas import tpu_sc as plsc

NC, NS, L = 2, 16, 16
NW = NC * NS
B, S, V, D = 4, 2048, 100000, 768
N = B * S
TPW = N // NW
C = 32
NCH = TPW // C
PR = C // B
NV = D // L
HR = 4
PEW = PR * D
AUX = PEW + C * 2 * L


def _body(idx_hbm, oidx_hbm, aux_hbm, segd_hbm, tok_hbm,
          out_hbm, idx_v, oidx_v, aux_v, segd_v, tok_v, res_v,
          tok_sem, aux_sem, out_sem, misc_sem):
    wid = lax.axis_index("s") * NC + lax.axis_index("c")

    cd_idx = pltpu.async_copy(idx_hbm.at[wid], idx_v, misc_sem)
    cd_oidx = pltpu.async_copy(oidx_hbm.at[wid], oidx_v, misc_sem)
    cd_segd = pltpu.async_copy(segd_hbm, segd_v, misc_sem)
    cd_idx.wait()

    def start_in(g):
        slot = lax.rem(g, 2)
        pltpu.async_copy(
            tok_hbm.at[idx_v.at[g]], tok_v.at[slot], tok_sem.at[slot])
        pltpu.async_copy(
            aux_hbm.at[wid, g], aux_v.at[slot], aux_sem.at[slot])

    def wait_in(g, slot):
        pltpu.make_async_copy(
            tok_hbm.at[idx_v.at[g]], tok_v.at[slot], tok_sem.at[slot]).wait()
        pltpu.make_async_copy(
            aux_hbm.at[wid, g], aux_v.at[slot], aux_sem.at[slot]).wait()

    def start_out(g, slot):
        pltpu.async_copy(
            res_v.at[slot], out_hbm.at[oidx_v.at[g]], out_sem.at[slot])

    def wait_out(g, slot):
        pltpu.make_async_copy(
            res_v.at[slot], out_hbm.at[oidx_v.at[g]], out_sem.at[slot]).wait()

    def compute(slot):
        for h in range(C // HR):
            r0 = h * HR
            was = tuple(
                aux_v[slot, pl.ds(PEW + (r0 + i) * 2 * L, L)]
                for i in range(HR))
            wbs = tuple(
                aux_v[slot, pl.ds(PEW + (r0 + i) * 2 * L + L, L)]
                for i in range(HR))

            def jbody(j, carry):
                was_, wbs_ = carry
                off = j * L
                a1 = segd_v[pl.ds(off, L)]
                a2 = segd_v[pl.ds(D + off, L)]
                for i in range(HR):
                    row = r0 + i
                    t = tok_v[slot, row, pl.ds(off, L)]
                    p = aux_v[slot, pl.ds((row // B) * D + off, L)]
                    res_v[slot, row, pl.ds(off, L)] = (
                        t + p + was_[i] * a1 + wbs_[i] * a2)
                return was_, wbs_

            plsc.parallel_loop(0, NV, 1, unroll=3, carry=(was, wbs))(jbody)

    start_in(0)
    start_in(1)
    cd_oidx.wait()
    cd_segd.wait()

    def gbody(g, carry):
        slot = lax.rem(g, 2)
        wait_in(g, slot)

        @pl.when(g >= 2)
        def _():
            wait_out(g - 2, slot)

        compute(slot)
        start_out(g, slot)

        @pl.when(g + 2 < NCH)
        def _():
            start_in(g + 2)

        return carry

    lax.fori_loop(0, NCH, gbody, 0)
    wait_out(NCH - 2, 0)
    wait_out(NCH - 1, 1)


_sc_call = functools.partial(
    pl.kernel,
    out_type=jax.ShapeDtypeStruct((N, D), jnp.float32),
    mesh=plsc.VectorSubcoreMesh(core_axis_name="c", subcore_axis_name="s"),
    scratch_types=[
        pltpu.VMEM((NCH, C), jnp.int32),
        pltpu.VMEM((NCH, C), jnp.int32),
        pltpu.VMEM((2, AUX), jnp.float32),
        pltpu.VMEM((2 * D,), jnp.float32),
        pltpu.VMEM((2, C, D), jnp.float32),
        pltpu.VMEM((2, C, D), jnp.float32),
        pltpu.SemaphoreType.DMA((2,)),
        pltpu.SemaphoreType.DMA((2,)),
        pltpu.SemaphoreType.DMA((2,)),
        pltpu.SemaphoreType.DMA,
    ],
)(_body)


def kernel(sequence, segment_labels, tok_table, seg_table, pe):
    seq_sm = sequence.T.reshape(NW, NCH, C).astype(jnp.int32)
    lbl_sm = segment_labels.T.reshape(NW, TPW).astype(jnp.int32)
    w = jnp.broadcast_to(
        jnp.stack([(lbl_sm >= 1), (lbl_sm >= 2)], axis=-1)
        .astype(jnp.float32)[..., None],
        (NW, TPW, 2, L)).reshape(NW, NCH, C * 2 * L)
    tp = jnp.arange(N, dtype=jnp.int32)
    oidx = ((tp % B) * S + tp // B).reshape(NW, NCH, C)
    segd = jnp.concatenate(
        [seg_table[1] - seg_table[0], seg_table[2] - seg_table[1]])
    pe5 = (pe.reshape(S, D) + seg_table[0]).reshape(NW, NCH, PEW)
    aux = jnp.concatenate([pe5, w], axis=-1)
    out = _sc_call(seq_sm, oidx, aux, segd, tok_table)
    return out.reshape(B, S, D)
